# Initial kernel scaffold; baseline (speedup 1.0000x reference)
import jax
import jax.numpy as jnp
from jax.experimental import pallas as pl


def kernel(x_seq, x_track, wih_f, b_f, wih_b, b_b, whh_f, wt, bt, w1, b1, w2p, b2p):
    raise NotImplementedError("write your pallas kernel here")


# speedup vs baseline: 2.7525x; 2.7525x over previous
from functools import partial

import jax
import jax.numpy as jnp
from jax.experimental import pallas as pl
from jax.experimental.pallas import tpu as pltpu


def _round_up(x, m):
    return ((x + m - 1) // m) * m


def _fused_bilstm_kernel(
    x_ref,
    xtr_ref,
    wihf_ref,
    bf_ref,
    wihb_ref,
    bb_ref,
    whhf_ref,
    wt_ref,
    btb_ref,
    w1_ref,
    b1_ref,
    w2_ref,
    b2_ref,
    out_ref,
    h_ref,
    c_ref,
    *,
    Tc: int,
    nc: int,
    H: int,
):
    k = pl.program_id(1)

    @pl.when(k == 0)
    def _init():
        h_ref[...] = jnp.zeros_like(h_ref)
        c_ref[...] = jnp.zeros_like(c_ref)

    wih = wihf_ref[...]
    whh = whhf_ref[...]
    b = bf_ref[...]

    h = h_ref[...]
    c = c_ref[...]
    for t in range(Tc):
        x_t = x_ref[:, t, :]
        gates = (
            jnp.dot(x_t, wih, preferred_element_type=jnp.float32)
            + jnp.dot(h, whh, preferred_element_type=jnp.float32)
            + b
        )
        ti = jnp.tanh(gates[:, 0:H] * 0.5)
        tf = jnp.tanh(gates[:, H:2 * H] * 0.5)
        g = jnp.tanh(gates[:, 2 * H:3 * H])
        to = jnp.tanh(gates[:, 3 * H:4 * H] * 0.5)
        c = 0.5 * ((1.0 + tf) * c + (1.0 + ti) * g)
        h = (0.5 * (1.0 + to)) * jnp.tanh(c)
    h_ref[...] = h
    c_ref[...] = c

    @pl.when(k == nc - 1)
    def _head():
        x_last = x_ref[:, Tc - 1, :]
        gb = (
            jnp.dot(x_last, wihb_ref[...], preferred_element_type=jnp.float32)
            + bb_ref[...]
        )
        ti_b = jnp.tanh(gb[:, 0:H] * 0.5)
        g_b = jnp.tanh(gb[:, 2 * H:3 * H])
        to_b = jnp.tanh(gb[:, 3 * H:4 * H] * 0.5)
        c_b = (0.5 * (1.0 + ti_b)) * g_b
        h_b = (0.5 * (1.0 + to_b)) * jnp.tanh(c_b)

        track = jnp.maximum(
            jnp.dot(xtr_ref[...], wt_ref[...], preferred_element_type=jnp.float32)
            + btb_ref[...],
            0.0,
        )

        pre = (
            jnp.dot(h, w1_ref[0:H, :], preferred_element_type=jnp.float32)
            + jnp.dot(h_b, w1_ref[H:2 * H, :], preferred_element_type=jnp.float32)
            + jnp.dot(track, w1_ref[2 * H:3 * H, :], preferred_element_type=jnp.float32)
            + b1_ref[...]
        )
        hidden = jnp.maximum(pre, 0.0)
        out = (
            jnp.dot(hidden, w2_ref[...], preferred_element_type=jnp.float32)
            + b2_ref[...]
        )
        out_ref[...] = out.astype(out_ref.dtype)


@jax.jit
def kernel(x_seq, x_track, wih_f, b_f, wih_b, b_b, whh_f, wt, bt, w1, b1, w2p, b2p):
    B, T, Din = x_seq.shape
    Dtrk = x_track.shape[1]
    H = whh_f.shape[0]

    B8 = _round_up(B, 8)
    btile = B8 // 2 if (B8 // 2) % 8 == 0 and B8 >= 16 else B8
    nb = B8 // btile
    B_pad = nb * btile
    if B_pad != B:
        x_seq = jnp.pad(x_seq, ((0, B_pad - B), (0, 0), (0, 0)))
        x_track = jnp.pad(x_track, ((0, B_pad - B), (0, 0)))

    Tc = 8 if T % 8 == 0 else T
    nc = T // Tc

    out = pl.pallas_call(
        partial(_fused_bilstm_kernel, Tc=Tc, nc=nc, H=H),
        out_shape=jax.ShapeDtypeStruct((B_pad, 128), jnp.float32),
        grid=(nb, nc),
        in_specs=[
            pl.BlockSpec((btile, Tc, Din), lambda i, k: (i, k, 0)),
            pl.BlockSpec((btile, Dtrk), lambda i, k: (i, 0)),
            pl.BlockSpec((Din, 4 * H), lambda i, k: (0, 0)),
            pl.BlockSpec((1, 4 * H), lambda i, k: (0, 0)),
            pl.BlockSpec((Din, 4 * H), lambda i, k: (0, 0)),
            pl.BlockSpec((1, 4 * H), lambda i, k: (0, 0)),
            pl.BlockSpec((H, 4 * H), lambda i, k: (0, 0)),
            pl.BlockSpec((Dtrk, H), lambda i, k: (0, 0)),
            pl.BlockSpec((1, H), lambda i, k: (0, 0)),
            pl.BlockSpec((3 * H, 64), lambda i, k: (0, 0)),
            pl.BlockSpec((1, 64), lambda i, k: (0, 0)),
            pl.BlockSpec((64, 128), lambda i, k: (0, 0)),
            pl.BlockSpec((1, 128), lambda i, k: (0, 0)),
        ],
        out_specs=pl.BlockSpec((btile, 128), lambda i, k: (i, 0)),
        scratch_shapes=[
            pltpu.VMEM((btile, H), jnp.float32),
            pltpu.VMEM((btile, H), jnp.float32),
        ],
        compiler_params=pltpu.CompilerParams(
            dimension_semantics=("parallel", "arbitrary"),
            vmem_limit_bytes=64 * 1024 * 1024,
        ),
    )(x_seq, x_track, wih_f, b_f, wih_b, b_b, whh_f, wt, bt, w1, b1, w2p, b2p)

    return out[:B, :3]
